# baseline (speedup 1.0000x reference)
import jax
import jax.numpy as jnp
from jax import lax
from jax.experimental import pallas as pl
from jax.experimental.pallas import tpu as pltpu
from jax.experimental.pallas import tpu_sc as plsc

N_BATCH = 4
SEQ = 4096
EMB = 2048
NC, NS = 2, 16
NW = NC * NS

SC_SEQ = 1024
S_PER_W = SC_SEQ // NW
C = 8
N_CHUNKS = S_PER_W // C
LANE_SL = EMB // 16

BLK_S = 512
SC_BLOCKS = SC_SEQ // BLK_S
TOTAL_BLOCKS = N_BATCH * (SEQ // BLK_S)
TC_BLOCKS = TOTAL_BLOCKS - SC_BLOCKS


def _sc_body(x_hbm, pos_hbm, out_hbm,
             bufp0, bufp1, bx0, bx1,
             psem0, psem1, lsem0, lsem1, ssem0, ssem1):
    wid = lax.axis_index("s") * NC + lax.axis_index("c")
    s_base = wid * S_PER_W

    bufp = (bufp0, bufp1)
    bx = (bx0, bx1)
    lsem = (lsem0, lsem1)
    ssem = (ssem0, ssem1)

    pltpu.async_copy(pos_hbm.at[pl.ds(s_base, C)], bufp0, psem0)
    pltpu.async_copy(x_hbm.at[0].at[pl.ds(s_base, C)], bx0, lsem0)

    def do_chunk(g, p):
        s0 = s_base + g * C

        pltpu.make_async_copy(
            pos_hbm.at[pl.ds(0, C)], bufp[p], (psem0, psem1)[p]
        ).wait()
        pltpu.make_async_copy(
            x_hbm.at[0].at[pl.ds(0, C)], bx[p], lsem[p]
        ).wait()

        @pl.when(g + 1 < N_CHUNKS)
        def _():
            pltpu.async_copy(
                pos_hbm.at[pl.ds(s0 + C, C)], bufp[1 - p], (psem0, psem1)[1 - p]
            )

            @pl.when(g > 0)
            def _():
                pltpu.make_async_copy(
                    bx[1 - p], out_hbm.at[0].at[pl.ds(0, C)], ssem[1 - p]
                ).wait()

            pltpu.async_copy(
                x_hbm.at[0].at[pl.ds(s0 + C, C)], bx[1 - p], lsem[1 - p]
            )

        for r in range(C):
            @plsc.parallel_loop(0, LANE_SL, unroll=8)
            def _(i, r=r):
                sl = pl.ds(i * 16, 16)
                bx[p][r, sl] = bx[p][r, sl] + bufp[p][r, sl]

        pltpu.async_copy(bx[p], out_hbm.at[0].at[pl.ds(s0, C)], ssem[p])

    def chunk_pair(g2, _):
        do_chunk(g2 * 2, 0)
        do_chunk(g2 * 2 + 1, 1)
        return 0

    lax.fori_loop(0, N_CHUNKS // 2, chunk_pair, 0)

    for b in range(2):
        pltpu.make_async_copy(
            bx[b], out_hbm.at[0].at[pl.ds(0, C)], ssem[b]
        ).wait()


def _tc_body(x_ref, pos_ref, o_ref):
    o_ref[...] = x_ref[...] + pos_ref[...]


def _merge_body(sc_ref, tc_ref, o_ref):
    del tc_ref
    o_ref[...] = sc_ref[...]


def kernel(x, pos_embedding):
    sc = pl.kernel(
        _sc_body,
        mesh=plsc.VectorSubcoreMesh(core_axis_name="c", subcore_axis_name="s"),
        out_type=jax.ShapeDtypeStruct((1, SC_SEQ, EMB), jnp.float32),
        scratch_types=(
            [pltpu.VMEM((C, EMB), jnp.float32) for _ in range(4)]
            + [pltpu.SemaphoreType.DMA for _ in range(6)]
        ),
    )
    sc_out = sc(x, pos_embedding)

    tc_out = pl.pallas_call(
        _tc_body,
        grid=(SEQ // BLK_S, N_BATCH),
        in_specs=[
            pl.BlockSpec((1, BLK_S, EMB), lambda s, b: (b, s, 0)),
            pl.BlockSpec((BLK_S, EMB), lambda s, b: (s, 0)),
        ],
        out_specs=pl.BlockSpec((1, BLK_S, EMB), lambda s, b: (b, s, 0)),
        out_shape=jax.ShapeDtypeStruct((N_BATCH, SEQ, EMB), x.dtype),
    )(x, pos_embedding)

    out = pl.pallas_call(
        _merge_body,
        grid=(SC_BLOCKS,),
        in_specs=[
            pl.BlockSpec((1, BLK_S, EMB), lambda i: (0, i, 0)),
            pl.BlockSpec(memory_space=pl.ANY),
        ],
        out_specs=pl.BlockSpec((1, BLK_S, EMB), lambda i: (0, i, 0)),
        out_shape=jax.ShapeDtypeStruct((N_BATCH, SEQ, EMB), x.dtype),
        input_output_aliases={1: 0},
    )(sc_out, tc_out)
    return out


# speedup vs baseline: 1.0625x; 1.0625x over previous
import jax
import jax.numpy as jnp
from jax import lax
from jax.experimental import pallas as pl
from jax.experimental.pallas import tpu as pltpu
from jax.experimental.pallas import tpu_sc as plsc

N_BATCH = 4
SEQ = 4096
EMB = 2048
NC, NS = 2, 16
NW = NC * NS

SC_SEQ = 512
S_PER_W = SC_SEQ // NW
C = 8
N_CHUNKS = S_PER_W // C
LANE_SL = EMB // 16

BLK_S = 512
SC_BLOCKS = SC_SEQ // BLK_S
TOTAL_BLOCKS = N_BATCH * (SEQ // BLK_S)
TC_BLOCKS = TOTAL_BLOCKS - SC_BLOCKS


def _sc_body(x_hbm, pos_hbm, out_hbm,
             bufp0, bufp1, bx0, bx1,
             psem0, psem1, lsem0, lsem1, ssem0, ssem1):
    wid = lax.axis_index("s") * NC + lax.axis_index("c")
    s_base = wid * S_PER_W

    bufp = (bufp0, bufp1)
    bx = (bx0, bx1)
    lsem = (lsem0, lsem1)
    ssem = (ssem0, ssem1)

    pltpu.async_copy(pos_hbm.at[pl.ds(s_base, C)], bufp0, psem0)
    pltpu.async_copy(x_hbm.at[0].at[pl.ds(s_base, C)], bx0, lsem0)

    def do_chunk(g, p):
        s0 = s_base + g * C

        pltpu.make_async_copy(
            pos_hbm.at[pl.ds(0, C)], bufp[p], (psem0, psem1)[p]
        ).wait()
        pltpu.make_async_copy(
            x_hbm.at[0].at[pl.ds(0, C)], bx[p], lsem[p]
        ).wait()

        @pl.when(g + 1 < N_CHUNKS)
        def _():
            pltpu.async_copy(
                pos_hbm.at[pl.ds(s0 + C, C)], bufp[1 - p], (psem0, psem1)[1 - p]
            )

            @pl.when(g > 0)
            def _():
                pltpu.make_async_copy(
                    bx[1 - p], out_hbm.at[0].at[pl.ds(0, C)], ssem[1 - p]
                ).wait()

            pltpu.async_copy(
                x_hbm.at[0].at[pl.ds(s0 + C, C)], bx[1 - p], lsem[1 - p]
            )

        for r in range(C):
            @plsc.parallel_loop(0, LANE_SL, unroll=8)
            def _(i, r=r):
                sl = pl.ds(i * 16, 16)
                bx[p][r, sl] = bx[p][r, sl] + bufp[p][r, sl]

        pltpu.async_copy(bx[p], out_hbm.at[0].at[pl.ds(s0, C)], ssem[p])

    def chunk_pair(g2, _):
        do_chunk(g2 * 2, 0)
        do_chunk(g2 * 2 + 1, 1)
        return 0

    lax.fori_loop(0, N_CHUNKS // 2, chunk_pair, 0)

    for b in range(2):
        pltpu.make_async_copy(
            bx[b], out_hbm.at[0].at[pl.ds(0, C)], ssem[b]
        ).wait()


def _tc_body(x_ref, pos_ref, o_ref):
    o_ref[...] = x_ref[...] + pos_ref[...]


def _merge_body(sc_ref, tc_ref, o_ref):
    del tc_ref
    o_ref[...] = sc_ref[...]


def kernel(x, pos_embedding):
    sc = pl.kernel(
        _sc_body,
        mesh=plsc.VectorSubcoreMesh(core_axis_name="c", subcore_axis_name="s"),
        out_type=jax.ShapeDtypeStruct((1, SC_SEQ, EMB), jnp.float32),
        scratch_types=(
            [pltpu.VMEM((C, EMB), jnp.float32) for _ in range(4)]
            + [pltpu.SemaphoreType.DMA for _ in range(6)]
        ),
    )
    sc_out = sc(x, pos_embedding)

    def _tc_batch(s, b):
        return jnp.where((b == 0) & (s < SC_BLOCKS), 1, b)

    tc_out = pl.pallas_call(
        _tc_body,
        grid=(SEQ // BLK_S, N_BATCH),
        in_specs=[
            pl.BlockSpec((1, BLK_S, EMB), lambda s, b: (_tc_batch(s, b), s, 0)),
            pl.BlockSpec((BLK_S, EMB), lambda s, b: (s, 0)),
        ],
        out_specs=pl.BlockSpec((1, BLK_S, EMB), lambda s, b: (_tc_batch(s, b), s, 0)),
        out_shape=jax.ShapeDtypeStruct((N_BATCH, SEQ, EMB), x.dtype),
    )(x, pos_embedding)

    out = pl.pallas_call(
        _merge_body,
        grid=(SC_BLOCKS,),
        in_specs=[
            pl.BlockSpec((1, BLK_S, EMB), lambda i: (0, i, 0)),
            pl.BlockSpec(memory_space=pl.ANY),
        ],
        out_specs=pl.BlockSpec((1, BLK_S, EMB), lambda i: (0, i, 0)),
        out_shape=jax.ShapeDtypeStruct((N_BATCH, SEQ, EMB), x.dtype),
        input_output_aliases={1: 0},
    )(sc_out, tc_out)
    return out


# speedup vs baseline: 1.0628x; 1.0003x over previous
import jax
import jax.numpy as jnp
from jax import lax
from jax.experimental import pallas as pl
from jax.experimental.pallas import tpu as pltpu
from jax.experimental.pallas import tpu_sc as plsc

N_BATCH = 4
SEQ = 4096
EMB = 2048
NC, NS = 2, 16
NW = NC * NS

SC_SEQ = 512
S_PER_W = SC_SEQ // NW
C = 8
N_CHUNKS = S_PER_W // C
LANE_SL = EMB // 16

BLK_S = 512
SC_BLOCKS = SC_SEQ // BLK_S


def _sc_body(x_hbm, pos_hbm, out_hbm,
             bufp0, bufp1, bx0, bx1,
             psem0, psem1, lsem0, lsem1, ssem0, ssem1):
    wid = lax.axis_index("s") * NC + lax.axis_index("c")
    s_base = wid * S_PER_W

    bufp = (bufp0, bufp1)
    bx = (bx0, bx1)
    lsem = (lsem0, lsem1)
    ssem = (ssem0, ssem1)

    pltpu.async_copy(pos_hbm.at[pl.ds(s_base, C)], bufp0, psem0)
    pltpu.async_copy(x_hbm.at[0].at[pl.ds(s_base, C)], bx0, lsem0)

    def do_chunk(g, p):
        s0 = s_base + g * C

        pltpu.make_async_copy(
            pos_hbm.at[pl.ds(0, C)], bufp[p], (psem0, psem1)[p]
        ).wait()
        pltpu.make_async_copy(
            x_hbm.at[0].at[pl.ds(0, C)], bx[p], lsem[p]
        ).wait()

        @pl.when(g + 1 < N_CHUNKS)
        def _():
            pltpu.async_copy(
                pos_hbm.at[pl.ds(s0 + C, C)], bufp[1 - p], (psem0, psem1)[1 - p]
            )

            @pl.when(g > 0)
            def _():
                pltpu.make_async_copy(
                    bx[1 - p], out_hbm.at[0].at[pl.ds(0, C)], ssem[1 - p]
                ).wait()

            pltpu.async_copy(
                x_hbm.at[0].at[pl.ds(s0 + C, C)], bx[1 - p], lsem[1 - p]
            )

        for r in range(C):
            @plsc.parallel_loop(0, LANE_SL, unroll=8)
            def _(i, r=r):
                sl = pl.ds(i * 16, 16)
                bx[p][r, sl] = bx[p][r, sl] + bufp[p][r, sl]

        pltpu.async_copy(bx[p], out_hbm.at[0].at[pl.ds(s0, C)], ssem[p])

    def chunk_pair(g2, _):
        do_chunk(g2 * 2, 0)
        do_chunk(g2 * 2 + 1, 1)
        return 0

    lax.fori_loop(0, N_CHUNKS // 2, chunk_pair, 0)

    for b in range(2):
        pltpu.make_async_copy(
            bx[b], out_hbm.at[0].at[pl.ds(0, C)], ssem[b]
        ).wait()


def _tc_body(x_ref, pos_ref, o_ref):
    o_ref[...] = x_ref[...] + pos_ref[...]


def _merge_body(sc_ref, tc_ref, o_ref):
    del tc_ref
    o_ref[...] = sc_ref[...]


def kernel(x, pos_embedding):
    sc = pl.kernel(
        _sc_body,
        mesh=plsc.VectorSubcoreMesh(core_axis_name="c", subcore_axis_name="s"),
        out_type=jax.ShapeDtypeStruct((1, SC_SEQ, EMB), jnp.float32),
        scratch_types=(
            [pltpu.VMEM((C, EMB), jnp.float32) for _ in range(4)]
            + [pltpu.SemaphoreType.DMA for _ in range(6)]
        ),
    )
    sc_out = sc(x, pos_embedding)

    def _tc_batch(s, b):
        return jnp.where((b == 0) & (s < SC_BLOCKS), 1, b)

    tc_out = pl.pallas_call(
        _tc_body,
        grid=(SEQ // BLK_S, N_BATCH),
        in_specs=[
            pl.BlockSpec((1, BLK_S, EMB), lambda s, b: (_tc_batch(s, b), s, 0)),
            pl.BlockSpec((BLK_S, EMB), lambda s, b: (s, 0)),
        ],
        out_specs=pl.BlockSpec((1, BLK_S, EMB), lambda s, b: (_tc_batch(s, b), s, 0)),
        out_shape=jax.ShapeDtypeStruct((N_BATCH, SEQ, EMB), x.dtype),
    )(x, pos_embedding)

    out = pl.pallas_call(
        _merge_body,
        grid=(SC_BLOCKS,),
        in_specs=[
            pl.BlockSpec((1, BLK_S, EMB), lambda i: (0, i, 0)),
            pl.BlockSpec(memory_space=pl.ANY),
        ],
        out_specs=pl.BlockSpec((1, BLK_S, EMB), lambda i: (0, i, 0)),
        out_shape=jax.ShapeDtypeStruct((N_BATCH, SEQ, EMB), x.dtype),
        input_output_aliases={1: 0},
    )(sc_out, tc_out)
    return out
